# baseline (speedup 1.0000x reference)
import numpy as np
import jax
import jax.numpy as jnp
from jax.experimental import pallas as pl
from jax.experimental.pallas import tpu as pltpu

_SIGMOID_LAMBDA = 28.0
_SIGMOID_PHI = 5.0
_CB = 8


def _probe_body(xf_ref, w_ref, g_ref, out_ref):
    out_ref[...] = g_ref[...] + w_ref[0, 0]


def _mac_body(xf_ref, w_ref, g_ref, out_ref):
    xf = xf_ref[...]
    s = jnp.sum(xf, axis=1, keepdims=True)
    rs = jnp.where(s > 0.0, 1.0 / s, 0.0)
    n = out_ref.shape[2]
    xb = xf.astype(jnp.bfloat16)
    wt = jnp.concatenate(
        [w_ref[c].astype(jnp.bfloat16) for c in range(_CB)], axis=1)
    y = jnp.dot(xb, wt, preferred_element_type=jnp.float32) * rs
    for c in range(_CB):
        t = jnp.log(1.0 / (1.0 + jnp.exp(
            -_SIGMOID_LAMBDA * y[:, c * n:(c + 1) * n] + _SIGMOID_PHI)))
        t = t + g_ref[:, c, :]
        m = jnp.max(t, axis=1, keepdims=True)
        iota = jax.lax.broadcasted_iota(jnp.int32, t.shape, 1)
        first = jnp.min(jnp.where(t == m, iota, n), axis=1, keepdims=True)
        out_ref[:, c, :] = (iota == first).astype(jnp.float32)


def kernel(x, weights):
    b = x.shape[0]
    num_cms, k, n = weights.shape
    xf = x.reshape(b, k)
    g = jax.random.gumbel(jax.random.key(123), (b, num_cms, n), jnp.float32)
    w2 = weights.reshape(num_cms * k * n // 512, 512)
    rows = k * n // 512
    return pl.pallas_call(
        _probe_body,
        grid=(num_cms // _CB,),
        in_specs=[
            pl.BlockSpec((b, k), lambda i: (0, 0)),
            pl.BlockSpec((_CB * rows, 512), lambda i: (i, 0)),
            pl.BlockSpec((b, _CB, n), lambda i: (0, i, 0)),
        ],
        out_specs=pl.BlockSpec((b, _CB, n), lambda i: (0, i, 0)),
        out_shape=jax.ShapeDtypeStruct((b, num_cms, n), jnp.float32),
        compiler_params=pltpu.CompilerParams(
            dimension_semantics=("arbitrary",),
            vmem_limit_bytes=100 * 1024 * 1024,
        ),
    )(xf, w2, g)


# speedup vs baseline: 5.9308x; 5.9308x over previous
import jax
import jax.numpy as jnp
from jax.experimental import pallas as pl
from jax.experimental.pallas import tpu as pltpu

_SIGMOID_LAMBDA = 28.0
_SIGMOID_PHI = 5.0
_CB = 8


def _mac_body(xt_ref, w_ref, g_ref, out_ref):
    xt = xt_ref[...]
    s = jnp.sum(xt, axis=0, keepdims=True)
    rs = jnp.where(s > 0.0, 1.0 / s, 0.0)
    cb, n, k = w_ref.shape
    b = xt.shape[1]
    wl = w_ref[...].reshape(cb * n, k).astype(jnp.bfloat16)
    yt = jnp.dot(wl, xt.astype(jnp.bfloat16),
                 preferred_element_type=jnp.float32) * rs
    t = jnp.log(1.0 / (1.0 + jnp.exp(-_SIGMOID_LAMBDA * yt + _SIGMOID_PHI)))
    t = (t + g_ref[...]).reshape(cb, n, b)
    m = jnp.max(t, axis=1, keepdims=True)
    iota = jax.lax.broadcasted_iota(jnp.int32, t.shape, 1)
    first = jnp.min(jnp.where(t == m, iota, n), axis=1, keepdims=True)
    out_ref[...] = (iota == first).astype(jnp.float32).reshape(cb * n, b)


def kernel(x, weights):
    b = x.shape[0]
    num_cms, k, n = weights.shape
    xt = x.reshape(b, k).T
    wt = weights.transpose(0, 2, 1)
    g = jax.random.gumbel(jax.random.key(123), (b, num_cms, n), jnp.float32)
    gt = g.transpose(1, 2, 0).reshape(num_cms * n, b)
    out_t = pl.pallas_call(
        _mac_body,
        grid=(num_cms // _CB,),
        in_specs=[
            pl.BlockSpec((k, b), lambda i: (0, 0)),
            pl.BlockSpec((_CB, n, k), lambda i: (i, 0, 0)),
            pl.BlockSpec((_CB * n, b), lambda i: (i, 0)),
        ],
        out_specs=pl.BlockSpec((_CB * n, b), lambda i: (i, 0)),
        out_shape=jax.ShapeDtypeStruct((num_cms * n, b), jnp.float32),
        compiler_params=pltpu.CompilerParams(
            dimension_semantics=("arbitrary",),
            vmem_limit_bytes=100 * 1024 * 1024,
        ),
    )(xt, wt, gt)
    return out_t.reshape(num_cms, n, b).transpose(2, 0, 1)


# speedup vs baseline: 5.9426x; 1.0020x over previous
import jax
import jax.numpy as jnp
from jax.experimental import pallas as pl
from jax.experimental.pallas import tpu as pltpu

_SIGMOID_LAMBDA = 28.0
_SIGMOID_PHI = 5.0
_CB = 8


def _mac_body(xt_ref, w_ref, g_ref, out_ref):
    xt = xt_ref[...]
    s = jnp.sum(xt, axis=0, keepdims=True)
    rs = jnp.where(s > 0.0, 1.0 / s, 0.0)
    cb, n, k = w_ref.shape
    b = xt.shape[1]
    wl = w_ref[...].reshape(cb * n, k)
    yt = jnp.dot(wl, xt, preferred_element_type=jnp.float32) * rs
    t = jnp.log(1.0 / (1.0 + jnp.exp(-_SIGMOID_LAMBDA * yt + _SIGMOID_PHI)))
    t = (t + g_ref[...]).reshape(cb, n, b)
    m = jnp.max(t, axis=1, keepdims=True)
    iota = jax.lax.broadcasted_iota(jnp.int32, t.shape, 1)
    first = jnp.min(jnp.where(t == m, iota, n), axis=1, keepdims=True)
    out_ref[...] = (iota == first).astype(jnp.float32).reshape(cb * n, b)


def kernel(x, weights):
    b = x.shape[0]
    num_cms, k, n = weights.shape
    xt = x.reshape(b, k).T
    wt = weights.transpose(0, 2, 1)
    g = jax.random.gumbel(jax.random.key(123), (b, num_cms, n), jnp.float32)
    gt = g.transpose(1, 2, 0).reshape(num_cms * n, b)
    out_t = pl.pallas_call(
        _mac_body,
        grid=(num_cms // _CB,),
        in_specs=[
            pl.BlockSpec((k, b), lambda i: (0, 0)),
            pl.BlockSpec((_CB, n, k), lambda i: (i, 0, 0)),
            pl.BlockSpec((_CB * n, b), lambda i: (i, 0)),
        ],
        out_specs=pl.BlockSpec((_CB * n, b), lambda i: (i, 0)),
        out_shape=jax.ShapeDtypeStruct((num_cms * n, b), jnp.float32),
        compiler_params=pltpu.CompilerParams(
            dimension_semantics=("arbitrary",),
            vmem_limit_bytes=100 * 1024 * 1024,
        ),
    )(xt, wt, gt)
    return out_t.reshape(num_cms, n, b).transpose(2, 0, 1)
